# baseline (speedup 1.0000x reference)
import functools

import jax
import jax.numpy as jnp
from jax import lax
from jax.experimental import pallas as pl
from jax.experimental.pallas import tpu as pltpu
from jax.experimental.pallas import tpu_sc as plsc

NUM_ROWS = 1_000_000
D = 64
BATCH = 16384
FEATS = 26
B_TOTAL = BATCH * FEATS
NC, NS = 2, 16
NW = NC * NS
B_PER_W = B_TOTAL // NW
IDX_W = 128
IDX_ROWS = B_PER_W // IDX_W
CHUNK_ROWS = 4
NCHUNK = IDX_ROWS // CHUNK_ROWS
CHUNK_B = CHUNK_ROWS * IDX_W


def _sc_gather(x_flat, table):
    mesh = plsc.VectorSubcoreMesh(core_axis_name="c", subcore_axis_name="s")

    @functools.partial(
        pl.kernel,
        out_type=jax.ShapeDtypeStruct((B_TOTAL, D), jnp.float32),
        mesh=mesh,
        scratch_types=[
            pltpu.VMEM((IDX_ROWS, IDX_W), jnp.int32),
            pltpu.VMEM((CHUNK_B, D), jnp.float32),
            pltpu.SemaphoreType.DMA,
        ],
        compiler_params=pltpu.CompilerParams(use_tc_tiling_on_sc=False),
    )
    def k(idx_hbm, table_hbm, out_hbm, idx_v, rows_v, gsem):
        wid = lax.axis_index("s") * NC + lax.axis_index("c")
        base = wid * B_PER_W
        pltpu.sync_copy(idx_hbm.at[wid], idx_v)

        def body(c, carry):
            cps = [
                pltpu.async_copy(
                    table_hbm.at[idx_v.at[c * CHUNK_ROWS + j]],
                    rows_v.at[pl.ds(j * IDX_W, IDX_W)],
                    gsem,
                )
                for j in range(CHUNK_ROWS)
            ]
            for cp in cps:
                cp.wait()
            pltpu.sync_copy(
                rows_v, out_hbm.at[pl.ds(base + c * CHUNK_B, CHUNK_B)]
            )
            return carry

        lax.fori_loop(0, NCHUNK, body, 0)

    return k(x_flat, table)


def kernel(x, table):
    x_flat = x.reshape(NW, IDX_ROWS, IDX_W)
    out = _sc_gather(x_flat, table)
    return out.reshape(BATCH, FEATS, D)


# speedup vs baseline: 1.0182x; 1.0182x over previous
import functools

import jax
import jax.numpy as jnp
from jax import lax
from jax.experimental import pallas as pl
from jax.experimental.pallas import tpu as pltpu
from jax.experimental.pallas import tpu_sc as plsc

NUM_ROWS = 1_000_000
D = 64
BATCH = 16384
FEATS = 26
B_TOTAL = BATCH * FEATS
NC, NS = 2, 16
NW = NC * NS
B_PER_W = B_TOTAL // NW
IDX_W = 128
IDX_ROWS = B_PER_W // IDX_W
CHUNK_ROWS = 2
NCHUNK = IDX_ROWS // CHUNK_ROWS
CHUNK_B = CHUNK_ROWS * IDX_W
NBUF = 4
NSTEADY = NCHUNK // NBUF - 1


def _sc_gather(x_flat, table):
    mesh = plsc.VectorSubcoreMesh(core_axis_name="c", subcore_axis_name="s")

    @functools.partial(
        pl.kernel,
        out_type=jax.ShapeDtypeStruct((B_TOTAL, D), jnp.float32),
        mesh=mesh,
        scratch_types=[
            pltpu.VMEM((IDX_ROWS, IDX_W), jnp.int32),
            pltpu.VMEM((NBUF, CHUNK_B, D), jnp.float32),
            [pltpu.SemaphoreType.DMA] * NBUF,
            [pltpu.SemaphoreType.DMA] * NBUF,
        ],
        compiler_params=pltpu.CompilerParams(use_tc_tiling_on_sc=False),
    )
    def k(idx_hbm, table_hbm, out_hbm, idx_v, rows_v, gsems, osems):
        wid = lax.axis_index("s") * NC + lax.axis_index("c")
        base = wid * B_PER_W
        pltpu.sync_copy(idx_hbm.at[wid], idx_v)

        def fire(c, b):
            for j in range(CHUNK_ROWS):
                pltpu.async_copy(
                    table_hbm.at[idx_v.at[c * CHUNK_ROWS + j]],
                    rows_v.at[b, pl.ds(j * IDX_W, IDX_W)],
                    gsems[b],
                )

        def wait_gathers(b):
            pltpu.make_async_copy(
                table_hbm.at[pl.ds(0, CHUNK_B)], rows_v.at[b], gsems[b]
            ).wait()

        def drain_out(b):
            pltpu.make_async_copy(
                rows_v.at[b], out_hbm.at[pl.ds(0, CHUNK_B)], osems[b]
            ).wait()

        for b in range(NBUF):
            fire(b, b)

        def body(t, carry):
            for b in range(NBUF):
                c = t * NBUF + b
                wait_gathers(b)
                pltpu.async_copy(
                    rows_v.at[b],
                    out_hbm.at[pl.ds(base + c * CHUNK_B, CHUNK_B)],
                    osems[b],
                )
                drain_out(b)
                fire(c + NBUF, b)
            return carry

        lax.fori_loop(0, NSTEADY, body, 0)

        for b in range(NBUF):
            c = NSTEADY * NBUF + b
            wait_gathers(b)
            pltpu.async_copy(
                rows_v.at[b],
                out_hbm.at[pl.ds(base + c * CHUNK_B, CHUNK_B)],
                osems[b],
            )
            drain_out(b)

    return k(x_flat, table)


def kernel(x, table):
    x_flat = x.reshape(NW, IDX_ROWS, IDX_W)
    out = _sc_gather(x_flat, table)
    return out.reshape(BATCH, FEATS, D)
